# baseline (speedup 1.0000x reference)
import functools

import jax
import jax.numpy as jnp
from jax import lax
from jax.experimental import pallas as pl
from jax.experimental.pallas import tpu as pltpu
from jax.experimental.pallas import tpu_sc as plsc

N = 10000
E = 320000
D_IN = 128
HID = 16
NCLS = 32

NC = 2
NS = 16
NP = 10240
SLICE = NP // NS
CB = 128
TOTCH = 2560
EP = TOTCH * CB
K = 8

CH0 = 1664
N0 = CH0 // NS
N1 = (TOTCH - CH0) // NS

RB = 1280
GRID = NP // RB

_mesh = plsc.VectorSubcoreMesh(core_axis_name="c", subcore_axis_name="s")


@functools.partial(
    pl.kernel,
    out_type=(
        jax.ShapeDtypeStruct((NC, NP), jnp.float32),
        jax.ShapeDtypeStruct((NC, NP), jnp.float32),
    ),
    mesh=_mesh,
    scratch_types=[
        pltpu.VMEM((2, 2, K, CB), jnp.int32),
        pltpu.VMEM((CB,), jnp.float32),
        pltpu.VMEM((SLICE,), jnp.float32),
        pltpu.VMEM_SHARED((NP,), jnp.float32),
        pltpu.VMEM_SHARED((NP,), jnp.float32),
        pltpu.SemaphoreType.DMA,
        pltpu.SemaphoreType.DMA,
    ],
)
def _deg(edges, dego_out, degi_out, idx_v, ones_v, z_v, dego_s, degi_s,
         ssem, isem):
    c = lax.axis_index("c")
    s = lax.axis_index("s")

    def fill_ones(i, carry):
        ones_v[pl.ds(i * 16, 16)] = jnp.full((16,), 1.0, jnp.float32)
        return carry

    lax.fori_loop(0, CB // 16, fill_ones, 0)

    def fill_zero(i, carry):
        z_v[pl.ds(i * 16, 16)] = jnp.zeros((16,), jnp.float32)
        return carry

    lax.fori_loop(0, SLICE // 16, fill_zero, 0)

    pltpu.sync_copy(z_v, dego_s.at[pl.ds(s * SLICE, SLICE)])
    pltpu.sync_copy(z_v, degi_s.at[pl.ds(s * SLICE, SLICE)])

    def drain_scat():
        pltpu.make_async_copy(dego_out.at[0, pl.ds(0, CB)], ones_v, ssem).wait()

    def load_idx(base, g, bank, sync):
        for h in range(2):
            src = edges.at[h, pl.ds(base + g * K, K)]
            if sync:
                pltpu.sync_copy(src, idx_v.at[bank, h])
            else:
                pltpu.async_copy(src, idx_v.at[bank, h], isem)

    def wait_idx():
        for _h in range(2):
            pltpu.make_async_copy(edges.at[0, pl.ds(0, K)], idx_v.at[0, 0],
                                  isem).wait()

    def run_worker(nch, base):
        ngrp = nch // K
        load_idx(base, 0, 0, True)
        plsc.subcore_barrier()

        def grp(g, carry):
            b = lax.rem(g, 2)

            @pl.when(g >= 1)
            def _():
                for _k in range(2 * K):
                    drain_scat()
                wait_idx()

            @pl.when(g + 1 < ngrp)
            def _():
                load_idx(base, g + 1, 1 - b, False)

            for k in range(K):
                pltpu.async_copy(ones_v, dego_s.at[idx_v.at[b, 0, k]],
                                 ssem, add=True)
                pltpu.async_copy(ones_v, degi_s.at[idx_v.at[b, 1, k]],
                                 ssem, add=True)
            return carry

        lax.fori_loop(0, ngrp, grp, 0)
        for _k in range(2 * K):
            drain_scat()
        plsc.subcore_barrier()

    @pl.when(c == 0)
    def _():
        run_worker(N0, s * N0)

    @pl.when(c == 1)
    def _():
        run_worker(N1, CH0 + s * N1)

    pltpu.sync_copy(dego_s.at[pl.ds(s * SLICE, SLICE)],
                    dego_out.at[c, pl.ds(s * SLICE, SLICE)])
    pltpu.sync_copy(degi_s.at[pl.ds(s * SLICE, SLICE)],
                    degi_out.at[c, pl.ds(s * SLICE, SLICE)])


def _make_mp(F, stage_table):
    scratch = [
        pltpu.VMEM((3, 2, K, CB), jnp.int32),
        pltpu.VMEM((2, K, CB, F), jnp.float32),
        pltpu.VMEM((64, F), jnp.float32),
        pltpu.VMEM_SHARED((NP, F), jnp.float32),
        pltpu.SemaphoreType.DMA,
        pltpu.SemaphoreType.DMA,
        pltpu.SemaphoreType.DMA,
    ]
    if stage_table:
        scratch.insert(3, pltpu.VMEM_SHARED((NP, F), jnp.float32))

    @functools.partial(
        pl.kernel,
        out_type=jax.ShapeDtypeStruct((NC, NP, F), jnp.float32),
        mesh=_mesh,
        scratch_types=scratch,
        compiler_params=pltpu.CompilerParams(use_tc_tiling_on_sc=False),
    )
    def _mp(table, edges, out, idx_v, msg_v, z_v, *rest):
        if stage_table:
            table_s, acc_s, gsem, ssem, isem = rest
        else:
            acc_s, gsem, ssem, isem = rest
            table_s = table
        c = lax.axis_index("c")
        s = lax.axis_index("s")

        def fill_zero(i, carry):
            for k in range(F // 16):
                z_v[i, pl.ds(k * 16, 16)] = jnp.zeros((16,), jnp.float32)
            return carry

        lax.fori_loop(0, 64, fill_zero, 0)

        if stage_table:
            pltpu.sync_copy(table.at[pl.ds(s * SLICE, SLICE)],
                            table_s.at[pl.ds(s * SLICE, SLICE)])
        for t in range(SLICE // 64):
            pltpu.async_copy(z_v, acc_s.at[pl.ds(s * SLICE + t * 64, 64)],
                             isem)
        for _t in range(SLICE // 64):
            pltpu.make_async_copy(table.at[pl.ds(0, 64)], z_v, isem).wait()

        def fire_gathers(gbank, mbank):
            for k in range(K):
                pltpu.async_copy(table_s.at[idx_v.at[gbank, 0, k]],
                                 msg_v.at[mbank, k], gsem)

        def fire_scatters(gbank, mbank):
            for k in range(K):
                pltpu.async_copy(msg_v.at[mbank, k],
                                 acc_s.at[idx_v.at[gbank, 1, k]],
                                 ssem, add=True)

        def drain(sem):
            pltpu.make_async_copy(table.at[pl.ds(0, CB)], msg_v.at[0, 0],
                                  sem).wait()

        def load_idx(base, g, bank, sync):
            for h in range(2):
                src = edges.at[h, pl.ds(base + g * K, K)]
                if sync:
                    pltpu.sync_copy(src, idx_v.at[bank, h])
                else:
                    pltpu.async_copy(src, idx_v.at[bank, h], isem)

        def wait_idx():
            for _h in range(2):
                pltpu.make_async_copy(edges.at[0, pl.ds(0, K)],
                                      idx_v.at[0, 0], isem).wait()

        def run_worker(nch, base):
            ngrp = nch // K
            load_idx(base, 0, 0, True)
            if ngrp > 1:
                load_idx(base, 1, 1, False)
            plsc.subcore_barrier()
            fire_gathers(0, 0)

            def grp(g, carry):
                b = lax.rem(g, 2)
                ib = lax.rem(g, 3)
                for _k in range(K):
                    drain(gsem)

                @pl.when(g >= 1)
                def _():
                    for _k in range(K):
                        drain(ssem)

                @pl.when(g + 1 < ngrp)
                def _():
                    wait_idx()

                @pl.when(g + 2 < ngrp)
                def _():
                    load_idx(base, g + 2, lax.rem(g + 2, 3), False)

                @pl.when(g + 1 < ngrp)
                def _():
                    fire_gathers(lax.rem(g + 1, 3), 1 - b)

                fire_scatters(ib, b)
                return carry

            lax.fori_loop(0, ngrp, grp, 0)
            for _k in range(K):
                drain(ssem)
            plsc.subcore_barrier()

        @pl.when(c == 0)
        def _():
            run_worker(N0, s * N0)

        @pl.when(c == 1)
        def _():
            run_worker(N1, CH0 + s * N1)

        pltpu.sync_copy(acc_s.at[pl.ds(s * SLICE, SLICE)],
                        out.at[c, pl.ds(s * SLICE, SLICE)])

    return _mp


_mp16 = _make_mp(HID, True)
_mp32 = _make_mp(NCLS, True)


def _tc_b_body(feat, w1, do0, do1, xs):
    nsrc = lax.rsqrt(jnp.maximum(do0[...] + do1[...], 1.0))
    xw = jnp.dot(feat[...], w1[...], preferred_element_type=jnp.float32)
    xs[...] = xw * nsrc


_tc_b = pl.pallas_call(
    _tc_b_body,
    out_shape=jax.ShapeDtypeStruct((NP, HID), jnp.float32),
    grid=(GRID,),
    in_specs=[
        pl.BlockSpec((RB, D_IN), lambda i: (i, 0)),
        pl.BlockSpec((D_IN, HID), lambda i: (0, 0)),
        pl.BlockSpec((RB, 1), lambda i: (i, 0)),
        pl.BlockSpec((RB, 1), lambda i: (i, 0)),
    ],
    out_specs=pl.BlockSpec((RB, HID), lambda i: (i, 0)),
)


def _tc_d_body(agg, di0, di1, do0, do1, b1r, w2, hs):
    ndst = lax.rsqrt(jnp.maximum(di0[...] + di1[...], 1.0))
    nsrc = lax.rsqrt(jnp.maximum(do0[...] + do1[...], 1.0))
    h = jnp.maximum((agg[0] + agg[1]) * ndst + b1r[...], 0.0)
    hw = jnp.dot(h, w2[...], preferred_element_type=jnp.float32)
    hs[...] = hw * nsrc


_tc_d = pl.pallas_call(
    _tc_d_body,
    out_shape=jax.ShapeDtypeStruct((NP, NCLS), jnp.float32),
    grid=(GRID,),
    in_specs=[
        pl.BlockSpec((NC, RB, HID), lambda i: (0, i, 0)),
        pl.BlockSpec((RB, 1), lambda i: (i, 0)),
        pl.BlockSpec((RB, 1), lambda i: (i, 0)),
        pl.BlockSpec((RB, 1), lambda i: (i, 0)),
        pl.BlockSpec((RB, 1), lambda i: (i, 0)),
        pl.BlockSpec((1, HID), lambda i: (0, 0)),
        pl.BlockSpec((HID, NCLS), lambda i: (0, 0)),
    ],
    out_specs=pl.BlockSpec((RB, NCLS), lambda i: (i, 0)),
)


def _tc_f_body(agg, di0, di1, b2r, out):
    ndst = lax.rsqrt(jnp.maximum(di0[...] + di1[...], 1.0))
    out[...] = (agg[0] + agg[1]) * ndst + b2r[...]


_tc_f = pl.pallas_call(
    _tc_f_body,
    out_shape=jax.ShapeDtypeStruct((NP, NCLS), jnp.float32),
    grid=(GRID,),
    in_specs=[
        pl.BlockSpec((NC, RB, NCLS), lambda i: (0, i, 0)),
        pl.BlockSpec((RB, 1), lambda i: (i, 0)),
        pl.BlockSpec((RB, 1), lambda i: (i, 0)),
        pl.BlockSpec((1, NCLS), lambda i: (0, 0)),
    ],
    out_specs=pl.BlockSpec((RB, NCLS), lambda i: (i, 0)),
)


def kernel(features, edge_index, W1, b1, W2, b2):
    featp = jnp.pad(features, ((0, NP - N), (0, 0)))
    ep = jnp.pad(edge_index, ((0, 0), (0, EP - E)), constant_values=NP - 1)
    edges = ep.reshape(2, TOTCH, CB)

    dego_p, degi_p = _deg(edges)
    do0 = dego_p[0].reshape(NP, 1)
    do1 = dego_p[1].reshape(NP, 1)
    di0 = degi_p[0].reshape(NP, 1)
    di1 = degi_p[1].reshape(NP, 1)

    xs = _tc_b(featp, W1, do0, do1)
    agg1 = _mp16(xs, edges)
    hs = _tc_d(agg1, di0, di1, do0, do1, b1.reshape(1, HID), W2)
    agg2 = _mp32(hs, edges)
    outp = _tc_f(agg2, di0, di1, b2.reshape(1, NCLS))
    return outp[:N]


# speedup vs baseline: 1.0281x; 1.0281x over previous
import functools

import jax
import jax.numpy as jnp
from jax import lax
from jax.experimental import pallas as pl
from jax.experimental.pallas import tpu as pltpu
from jax.experimental.pallas import tpu_sc as plsc

N = 10000
E = 320000
D_IN = 128
HID = 16
NCLS = 32

NC = 2
NS = 16
NP = 10240
SLICE = NP // NS
CB = 128
TOTCH = 2560
EP = TOTCH * CB
K = 8

CH0 = 1536
N0 = CH0 // NS
N1 = (TOTCH - CH0) // NS

RB = 5120
GRID = NP // RB

_mesh = plsc.VectorSubcoreMesh(core_axis_name="c", subcore_axis_name="s")


@functools.partial(
    pl.kernel,
    out_type=(
        jax.ShapeDtypeStruct((NC, NP), jnp.float32),
        jax.ShapeDtypeStruct((NC, NP), jnp.float32),
    ),
    mesh=_mesh,
    scratch_types=[
        pltpu.VMEM((2, 2, K, CB), jnp.int32),
        pltpu.VMEM((CB,), jnp.float32),
        pltpu.VMEM((SLICE,), jnp.float32),
        pltpu.VMEM_SHARED((NP,), jnp.float32),
        pltpu.VMEM_SHARED((NP,), jnp.float32),
        pltpu.SemaphoreType.DMA,
        pltpu.SemaphoreType.DMA,
    ],
)
def _deg(edges, dego_out, degi_out, idx_v, ones_v, z_v, dego_s, degi_s,
         ssem, isem):
    c = lax.axis_index("c")
    s = lax.axis_index("s")

    def fill_ones(i, carry):
        ones_v[pl.ds(i * 16, 16)] = jnp.full((16,), 1.0, jnp.float32)
        return carry

    lax.fori_loop(0, CB // 16, fill_ones, 0)

    def fill_zero(i, carry):
        z_v[pl.ds(i * 16, 16)] = jnp.zeros((16,), jnp.float32)
        return carry

    lax.fori_loop(0, SLICE // 16, fill_zero, 0)

    pltpu.sync_copy(z_v, dego_s.at[pl.ds(s * SLICE, SLICE)])
    pltpu.sync_copy(z_v, degi_s.at[pl.ds(s * SLICE, SLICE)])

    def drain_scat():
        pltpu.make_async_copy(dego_out.at[0, pl.ds(0, CB)], ones_v, ssem).wait()

    def load_idx(base, g, bank, sync):
        for h in range(2):
            src = edges.at[h, pl.ds(base + g * K, K)]
            if sync:
                pltpu.sync_copy(src, idx_v.at[bank, h])
            else:
                pltpu.async_copy(src, idx_v.at[bank, h], isem)

    def wait_idx():
        for _h in range(2):
            pltpu.make_async_copy(edges.at[0, pl.ds(0, K)], idx_v.at[0, 0],
                                  isem).wait()

    def run_worker(nch, base):
        ngrp = nch // K
        load_idx(base, 0, 0, True)
        plsc.subcore_barrier()

        def grp(g, carry):
            b = lax.rem(g, 2)

            @pl.when(g >= 1)
            def _():
                for _k in range(2 * K):
                    drain_scat()
                wait_idx()

            @pl.when(g + 1 < ngrp)
            def _():
                load_idx(base, g + 1, 1 - b, False)

            for k in range(K):
                pltpu.async_copy(ones_v, dego_s.at[idx_v.at[b, 0, k]],
                                 ssem, add=True)
                pltpu.async_copy(ones_v, degi_s.at[idx_v.at[b, 1, k]],
                                 ssem, add=True)
            return carry

        lax.fori_loop(0, ngrp, grp, 0)
        for _k in range(2 * K):
            drain_scat()
        plsc.subcore_barrier()

    @pl.when(c == 0)
    def _():
        run_worker(N0, s * N0)

    @pl.when(c == 1)
    def _():
        run_worker(N1, CH0 + s * N1)

    pltpu.sync_copy(dego_s.at[pl.ds(s * SLICE, SLICE)],
                    dego_out.at[c, pl.ds(s * SLICE, SLICE)])
    pltpu.sync_copy(degi_s.at[pl.ds(s * SLICE, SLICE)],
                    degi_out.at[c, pl.ds(s * SLICE, SLICE)])


def _make_mp(F, stage_table):
    scratch = [
        pltpu.VMEM((3, 2, K, CB), jnp.int32),
        pltpu.VMEM((2, K, CB, F), jnp.float32),
        pltpu.VMEM((64, F), jnp.float32),
        pltpu.VMEM_SHARED((NP, F), jnp.float32),
        pltpu.SemaphoreType.DMA,
        pltpu.SemaphoreType.DMA,
        pltpu.SemaphoreType.DMA,
    ]
    if stage_table:
        scratch.insert(3, pltpu.VMEM_SHARED((NP, F), jnp.float32))

    @functools.partial(
        pl.kernel,
        out_type=jax.ShapeDtypeStruct((NC, NP, F), jnp.float32),
        mesh=_mesh,
        scratch_types=scratch,
        compiler_params=pltpu.CompilerParams(use_tc_tiling_on_sc=False),
    )
    def _mp(table, edges, out, idx_v, msg_v, z_v, *rest):
        if stage_table:
            table_s, acc_s, gsem, ssem, isem = rest
        else:
            acc_s, gsem, ssem, isem = rest
            table_s = table
        c = lax.axis_index("c")
        s = lax.axis_index("s")

        def fill_zero(i, carry):
            for k in range(F // 16):
                z_v[i, pl.ds(k * 16, 16)] = jnp.zeros((16,), jnp.float32)
            return carry

        lax.fori_loop(0, 64, fill_zero, 0)

        if stage_table:
            pltpu.sync_copy(table.at[pl.ds(s * SLICE, SLICE)],
                            table_s.at[pl.ds(s * SLICE, SLICE)])
        for t in range(SLICE // 64):
            pltpu.async_copy(z_v, acc_s.at[pl.ds(s * SLICE + t * 64, 64)],
                             isem)
        for _t in range(SLICE // 64):
            pltpu.make_async_copy(table.at[pl.ds(0, 64)], z_v, isem).wait()

        def fire_gathers(gbank, mbank):
            for k in range(K):
                pltpu.async_copy(table_s.at[idx_v.at[gbank, 0, k]],
                                 msg_v.at[mbank, k], gsem)

        def fire_scatters(gbank, mbank):
            for k in range(K):
                pltpu.async_copy(msg_v.at[mbank, k],
                                 acc_s.at[idx_v.at[gbank, 1, k]],
                                 ssem, add=True)

        def drain(sem):
            pltpu.make_async_copy(table.at[pl.ds(0, CB)], msg_v.at[0, 0],
                                  sem).wait()

        def load_idx(base, g, bank, sync):
            for h in range(2):
                src = edges.at[h, pl.ds(base + g * K, K)]
                if sync:
                    pltpu.sync_copy(src, idx_v.at[bank, h])
                else:
                    pltpu.async_copy(src, idx_v.at[bank, h], isem)

        def wait_idx():
            for _h in range(2):
                pltpu.make_async_copy(edges.at[0, pl.ds(0, K)],
                                      idx_v.at[0, 0], isem).wait()

        def run_worker(nch, base):
            ngrp = nch // K
            load_idx(base, 0, 0, True)
            if ngrp > 1:
                load_idx(base, 1, 1, False)
            plsc.subcore_barrier()
            fire_gathers(0, 0)

            def grp(g, carry):
                b = lax.rem(g, 2)
                ib = lax.rem(g, 3)
                for _k in range(K):
                    drain(gsem)

                @pl.when(g >= 1)
                def _():
                    for _k in range(K):
                        drain(ssem)

                @pl.when(g + 1 < ngrp)
                def _():
                    wait_idx()

                @pl.when(g + 2 < ngrp)
                def _():
                    load_idx(base, g + 2, lax.rem(g + 2, 3), False)

                @pl.when(g + 1 < ngrp)
                def _():
                    fire_gathers(lax.rem(g + 1, 3), 1 - b)

                fire_scatters(ib, b)
                return carry

            lax.fori_loop(0, ngrp, grp, 0)
            for _k in range(K):
                drain(ssem)
            plsc.subcore_barrier()

        @pl.when(c == 0)
        def _():
            run_worker(N0, s * N0)

        @pl.when(c == 1)
        def _():
            run_worker(N1, CH0 + s * N1)

        pltpu.sync_copy(acc_s.at[pl.ds(s * SLICE, SLICE)],
                        out.at[c, pl.ds(s * SLICE, SLICE)])

    return _mp


_mp16 = _make_mp(HID, True)
_mp32 = _make_mp(NCLS, True)


def _tc_b_body(feat, w1, do0, do1, xs):
    nsrc = lax.rsqrt(jnp.maximum(do0[...] + do1[...], 1.0))
    xw = jnp.dot(feat[...], w1[...], preferred_element_type=jnp.float32)
    xs[...] = xw * nsrc


_tc_b = pl.pallas_call(
    _tc_b_body,
    out_shape=jax.ShapeDtypeStruct((NP, HID), jnp.float32),
    grid=(GRID,),
    in_specs=[
        pl.BlockSpec((RB, D_IN), lambda i: (i, 0)),
        pl.BlockSpec((D_IN, HID), lambda i: (0, 0)),
        pl.BlockSpec((RB, 1), lambda i: (i, 0)),
        pl.BlockSpec((RB, 1), lambda i: (i, 0)),
    ],
    out_specs=pl.BlockSpec((RB, HID), lambda i: (i, 0)),
)


def _tc_d_body(agg, di0, di1, do0, do1, b1r, w2, hs):
    ndst = lax.rsqrt(jnp.maximum(di0[...] + di1[...], 1.0))
    nsrc = lax.rsqrt(jnp.maximum(do0[...] + do1[...], 1.0))
    h = jnp.maximum((agg[0] + agg[1]) * ndst + b1r[...], 0.0)
    hw = jnp.dot(h, w2[...], preferred_element_type=jnp.float32)
    hs[...] = hw * nsrc


_tc_d = pl.pallas_call(
    _tc_d_body,
    out_shape=jax.ShapeDtypeStruct((NP, NCLS), jnp.float32),
    grid=(GRID,),
    in_specs=[
        pl.BlockSpec((NC, RB, HID), lambda i: (0, i, 0)),
        pl.BlockSpec((RB, 1), lambda i: (i, 0)),
        pl.BlockSpec((RB, 1), lambda i: (i, 0)),
        pl.BlockSpec((RB, 1), lambda i: (i, 0)),
        pl.BlockSpec((RB, 1), lambda i: (i, 0)),
        pl.BlockSpec((1, HID), lambda i: (0, 0)),
        pl.BlockSpec((HID, NCLS), lambda i: (0, 0)),
    ],
    out_specs=pl.BlockSpec((RB, NCLS), lambda i: (i, 0)),
)


def _tc_f_body(agg, di0, di1, b2r, out):
    ndst = lax.rsqrt(jnp.maximum(di0[...] + di1[...], 1.0))
    out[...] = (agg[0] + agg[1]) * ndst + b2r[...]


_tc_f = pl.pallas_call(
    _tc_f_body,
    out_shape=jax.ShapeDtypeStruct((NP, NCLS), jnp.float32),
    grid=(GRID,),
    in_specs=[
        pl.BlockSpec((NC, RB, NCLS), lambda i: (0, i, 0)),
        pl.BlockSpec((RB, 1), lambda i: (i, 0)),
        pl.BlockSpec((RB, 1), lambda i: (i, 0)),
        pl.BlockSpec((1, NCLS), lambda i: (0, 0)),
    ],
    out_specs=pl.BlockSpec((RB, NCLS), lambda i: (i, 0)),
)


def kernel(features, edge_index, W1, b1, W2, b2):
    featp = jnp.pad(features, ((0, NP - N), (0, 0)))
    ep = jnp.pad(edge_index, ((0, 0), (0, EP - E)), constant_values=NP - 1)
    edges = ep.reshape(2, TOTCH, CB)

    dego_p, degi_p = _deg(edges)
    do0 = dego_p[0].reshape(NP, 1)
    do1 = dego_p[1].reshape(NP, 1)
    di0 = degi_p[0].reshape(NP, 1)
    di1 = degi_p[1].reshape(NP, 1)

    xs = _tc_b(featp, W1, do0, do1)
    agg1 = _mp16(xs, edges)
    hs = _tc_d(agg1, di0, di1, do0, do1, b1.reshape(1, HID), W2)
    agg2 = _mp32(hs, edges)
    outp = _tc_f(agg2, di0, di1, b2.reshape(1, NCLS))
    return outp[:N]


# speedup vs baseline: 1.0354x; 1.0071x over previous
import functools

import jax
import jax.numpy as jnp
from jax import lax
from jax.experimental import pallas as pl
from jax.experimental.pallas import tpu as pltpu
from jax.experimental.pallas import tpu_sc as plsc

N = 10000
E = 320000
D_IN = 128
HID = 16
NCLS = 32

NC = 2
NS = 16
NP = 10240
SLICE = NP // NS
CB = 128
TOTCH = 2560
EP = TOTCH * CB
K = 8

CH0 = 1536
N0 = CH0 // NS
N1 = (TOTCH - CH0) // NS
CH0D = 1792
N0D = CH0D // NS
N1D = (TOTCH - CH0D) // NS

RB = 5120
GRID = NP // RB

_mesh = plsc.VectorSubcoreMesh(core_axis_name="c", subcore_axis_name="s")


@functools.partial(
    pl.kernel,
    out_type=(
        jax.ShapeDtypeStruct((NC, NP), jnp.float32),
        jax.ShapeDtypeStruct((NC, NP), jnp.float32),
    ),
    mesh=_mesh,
    scratch_types=[
        pltpu.VMEM((2, 2, K, CB), jnp.int32),
        pltpu.VMEM((CB,), jnp.float32),
        pltpu.VMEM((SLICE,), jnp.float32),
        pltpu.VMEM_SHARED((NP,), jnp.float32),
        pltpu.VMEM_SHARED((NP,), jnp.float32),
        pltpu.SemaphoreType.DMA,
        pltpu.SemaphoreType.DMA,
    ],
)
def _deg(edges, dego_out, degi_out, idx_v, ones_v, z_v, dego_s, degi_s,
         ssem, isem):
    c = lax.axis_index("c")
    s = lax.axis_index("s")

    def fill_ones(i, carry):
        ones_v[pl.ds(i * 16, 16)] = jnp.full((16,), 1.0, jnp.float32)
        return carry

    lax.fori_loop(0, CB // 16, fill_ones, 0)

    def fill_zero(i, carry):
        z_v[pl.ds(i * 16, 16)] = jnp.zeros((16,), jnp.float32)
        return carry

    lax.fori_loop(0, SLICE // 16, fill_zero, 0)

    pltpu.sync_copy(z_v, dego_s.at[pl.ds(s * SLICE, SLICE)])
    pltpu.sync_copy(z_v, degi_s.at[pl.ds(s * SLICE, SLICE)])

    def drain_scat():
        pltpu.make_async_copy(dego_out.at[0, pl.ds(0, CB)], ones_v, ssem).wait()

    def load_idx(base, g, bank, sync):
        for h in range(2):
            src = edges.at[h, pl.ds(base + g * K, K)]
            if sync:
                pltpu.sync_copy(src, idx_v.at[bank, h])
            else:
                pltpu.async_copy(src, idx_v.at[bank, h], isem)

    def wait_idx():
        for _h in range(2):
            pltpu.make_async_copy(edges.at[0, pl.ds(0, K)], idx_v.at[0, 0],
                                  isem).wait()

    def run_worker(nch, base):
        ngrp = nch // K
        load_idx(base, 0, 0, True)
        plsc.subcore_barrier()

        def grp(g, carry):
            b = lax.rem(g, 2)

            @pl.when(g >= 1)
            def _():
                for _k in range(2 * K):
                    drain_scat()
                wait_idx()

            @pl.when(g + 1 < ngrp)
            def _():
                load_idx(base, g + 1, 1 - b, False)

            for k in range(K):
                pltpu.async_copy(ones_v, dego_s.at[idx_v.at[b, 0, k]],
                                 ssem, add=True)
                pltpu.async_copy(ones_v, degi_s.at[idx_v.at[b, 1, k]],
                                 ssem, add=True)
            return carry

        lax.fori_loop(0, ngrp, grp, 0)
        for _k in range(2 * K):
            drain_scat()
        plsc.subcore_barrier()

    @pl.when(c == 0)
    def _():
        run_worker(N0D, s * N0D)

    @pl.when(c == 1)
    def _():
        run_worker(N1D, CH0D + s * N1D)

    pltpu.sync_copy(dego_s.at[pl.ds(s * SLICE, SLICE)],
                    dego_out.at[c, pl.ds(s * SLICE, SLICE)])
    pltpu.sync_copy(degi_s.at[pl.ds(s * SLICE, SLICE)],
                    degi_out.at[c, pl.ds(s * SLICE, SLICE)])


def _make_mp(F, stage_table):
    scratch = [
        pltpu.VMEM((3, 2, K, CB), jnp.int32),
        pltpu.VMEM((2, K, CB, F), jnp.float32),
        pltpu.VMEM((64, F), jnp.float32),
        pltpu.VMEM_SHARED((NP, F), jnp.float32),
        pltpu.SemaphoreType.DMA,
        pltpu.SemaphoreType.DMA,
        pltpu.SemaphoreType.DMA,
    ]
    if stage_table:
        scratch.insert(3, pltpu.VMEM_SHARED((NP, F), jnp.float32))

    @functools.partial(
        pl.kernel,
        out_type=jax.ShapeDtypeStruct((NC, NP, F), jnp.float32),
        mesh=_mesh,
        scratch_types=scratch,
        compiler_params=pltpu.CompilerParams(use_tc_tiling_on_sc=False),
    )
    def _mp(table, edges, out, idx_v, msg_v, z_v, *rest):
        if stage_table:
            table_s, acc_s, gsem, ssem, isem = rest
        else:
            acc_s, gsem, ssem, isem = rest
            table_s = table
        c = lax.axis_index("c")
        s = lax.axis_index("s")

        def fill_zero(i, carry):
            for k in range(F // 16):
                z_v[i, pl.ds(k * 16, 16)] = jnp.zeros((16,), jnp.float32)
            return carry

        lax.fori_loop(0, 64, fill_zero, 0)

        if stage_table:
            pltpu.sync_copy(table.at[pl.ds(s * SLICE, SLICE)],
                            table_s.at[pl.ds(s * SLICE, SLICE)])
        for t in range(SLICE // 64):
            pltpu.async_copy(z_v, acc_s.at[pl.ds(s * SLICE + t * 64, 64)],
                             isem)
        for _t in range(SLICE // 64):
            pltpu.make_async_copy(table.at[pl.ds(0, 64)], z_v, isem).wait()

        def fire_gathers(gbank, mbank):
            for k in range(K):
                pltpu.async_copy(table_s.at[idx_v.at[gbank, 0, k]],
                                 msg_v.at[mbank, k], gsem)

        def fire_scatters(gbank, mbank):
            for k in range(K):
                pltpu.async_copy(msg_v.at[mbank, k],
                                 acc_s.at[idx_v.at[gbank, 1, k]],
                                 ssem, add=True)

        def drain(sem):
            pltpu.make_async_copy(table.at[pl.ds(0, CB)], msg_v.at[0, 0],
                                  sem).wait()

        def load_idx(base, g, bank, sync):
            for h in range(2):
                src = edges.at[h, pl.ds(base + g * K, K)]
                if sync:
                    pltpu.sync_copy(src, idx_v.at[bank, h])
                else:
                    pltpu.async_copy(src, idx_v.at[bank, h], isem)

        def wait_idx():
            for _h in range(2):
                pltpu.make_async_copy(edges.at[0, pl.ds(0, K)],
                                      idx_v.at[0, 0], isem).wait()

        def run_worker(nch, base):
            ngrp = nch // K
            load_idx(base, 0, 0, True)
            if ngrp > 1:
                load_idx(base, 1, 1, False)
            plsc.subcore_barrier()
            fire_gathers(0, 0)

            def grp(g, carry):
                b = lax.rem(g, 2)
                ib = lax.rem(g, 3)
                for _k in range(K):
                    drain(gsem)

                @pl.when(g >= 1)
                def _():
                    for _k in range(K):
                        drain(ssem)

                @pl.when(g + 1 < ngrp)
                def _():
                    wait_idx()

                @pl.when(g + 2 < ngrp)
                def _():
                    load_idx(base, g + 2, lax.rem(g + 2, 3), False)

                @pl.when(g + 1 < ngrp)
                def _():
                    fire_gathers(lax.rem(g + 1, 3), 1 - b)

                fire_scatters(ib, b)
                return carry

            lax.fori_loop(0, ngrp, grp, 0)
            for _k in range(K):
                drain(ssem)
            plsc.subcore_barrier()

        @pl.when(c == 0)
        def _():
            run_worker(N0, s * N0)

        @pl.when(c == 1)
        def _():
            run_worker(N1, CH0 + s * N1)

        pltpu.sync_copy(acc_s.at[pl.ds(s * SLICE, SLICE)],
                        out.at[c, pl.ds(s * SLICE, SLICE)])

    return _mp


_mp16 = _make_mp(HID, True)
_mp32 = _make_mp(NCLS, True)


def _tc_b_body(feat, w1, do0, do1, xs):
    nsrc = lax.rsqrt(jnp.maximum(do0[...] + do1[...], 1.0))
    xw = jnp.dot(feat[...], w1[...], preferred_element_type=jnp.float32)
    xs[...] = xw * nsrc


_tc_b = pl.pallas_call(
    _tc_b_body,
    out_shape=jax.ShapeDtypeStruct((NP, HID), jnp.float32),
    grid=(GRID,),
    in_specs=[
        pl.BlockSpec((RB, D_IN), lambda i: (i, 0)),
        pl.BlockSpec((D_IN, HID), lambda i: (0, 0)),
        pl.BlockSpec((RB, 1), lambda i: (i, 0)),
        pl.BlockSpec((RB, 1), lambda i: (i, 0)),
    ],
    out_specs=pl.BlockSpec((RB, HID), lambda i: (i, 0)),
)


def _tc_d_body(agg, di0, di1, do0, do1, b1r, w2, hs):
    ndst = lax.rsqrt(jnp.maximum(di0[...] + di1[...], 1.0))
    nsrc = lax.rsqrt(jnp.maximum(do0[...] + do1[...], 1.0))
    h = jnp.maximum((agg[0] + agg[1]) * ndst + b1r[...], 0.0)
    hw = jnp.dot(h, w2[...], preferred_element_type=jnp.float32)
    hs[...] = hw * nsrc


_tc_d = pl.pallas_call(
    _tc_d_body,
    out_shape=jax.ShapeDtypeStruct((NP, NCLS), jnp.float32),
    grid=(GRID,),
    in_specs=[
        pl.BlockSpec((NC, RB, HID), lambda i: (0, i, 0)),
        pl.BlockSpec((RB, 1), lambda i: (i, 0)),
        pl.BlockSpec((RB, 1), lambda i: (i, 0)),
        pl.BlockSpec((RB, 1), lambda i: (i, 0)),
        pl.BlockSpec((RB, 1), lambda i: (i, 0)),
        pl.BlockSpec((1, HID), lambda i: (0, 0)),
        pl.BlockSpec((HID, NCLS), lambda i: (0, 0)),
    ],
    out_specs=pl.BlockSpec((RB, NCLS), lambda i: (i, 0)),
)


def _tc_f_body(agg, di0, di1, b2r, out):
    ndst = lax.rsqrt(jnp.maximum(di0[...] + di1[...], 1.0))
    out[...] = (agg[0] + agg[1]) * ndst + b2r[...]


_tc_f = pl.pallas_call(
    _tc_f_body,
    out_shape=jax.ShapeDtypeStruct((NP, NCLS), jnp.float32),
    grid=(GRID,),
    in_specs=[
        pl.BlockSpec((NC, RB, NCLS), lambda i: (0, i, 0)),
        pl.BlockSpec((RB, 1), lambda i: (i, 0)),
        pl.BlockSpec((RB, 1), lambda i: (i, 0)),
        pl.BlockSpec((1, NCLS), lambda i: (0, 0)),
    ],
    out_specs=pl.BlockSpec((RB, NCLS), lambda i: (i, 0)),
)


def kernel(features, edge_index, W1, b1, W2, b2):
    featp = jnp.pad(features, ((0, NP - N), (0, 0)))
    ep = jnp.pad(edge_index, ((0, 0), (0, EP - E)), constant_values=NP - 1)
    edges = ep.reshape(2, TOTCH, CB)

    dego_p, degi_p = _deg(edges)
    do0 = dego_p[0].reshape(NP, 1)
    do1 = dego_p[1].reshape(NP, 1)
    di0 = degi_p[0].reshape(NP, 1)
    di1 = degi_p[1].reshape(NP, 1)

    xs = _tc_b(featp, W1, do0, do1)
    agg1 = _mp16(xs, edges)
    hs = _tc_d(agg1, di0, di1, do0, do1, b1.reshape(1, HID), W2)
    agg2 = _mp32(hs, edges)
    outp = _tc_f(agg2, di0, di1, b2.reshape(1, NCLS))
    return outp[:N]
